# Initial kernel scaffold; baseline (speedup 1.0000x reference)
import jax
import jax.numpy as jnp
from jax.experimental import pallas as pl


def kernel(boxes, scores):
    raise NotImplementedError("write your pallas kernel here")


# speedup vs baseline: 1.7536x; 1.7536x over previous
import jax
import jax.numpy as jnp
from jax.experimental import pallas as pl
from jax.experimental.pallas import tpu as pltpu

_N = 5000
_NPAD = 5120
_BI = 256
_IOU_THRESH = 0.5


def _nms_block_kernel(row_ref, col_ref, out_ref):
    i = pl.program_id(0)
    rows = row_ref[:, :]
    rx1 = rows[:, 0:1]
    ry1 = rows[:, 1:2]
    rx2 = rows[:, 2:3]
    ry2 = rows[:, 3:4]
    rs = rows[:, 4:5]

    cols = col_ref[:, :]
    cx1 = cols[0:1, :]
    cy1 = cols[1:2, :]
    cx2 = cols[2:3, :]
    cy2 = cols[3:4, :]
    cs = cols[4:5, :]

    r_area = (rx2 - rx1) * (ry2 - ry1)
    c_area = (cx2 - cx1) * (cy2 - cy1)
    ltx = jnp.maximum(rx1, cx1)
    lty = jnp.maximum(ry1, cy1)
    rbx = jnp.minimum(rx2, cx2)
    rby = jnp.minimum(ry2, cy2)
    w = jnp.clip(rbx - ltx, 0.0, None)
    h = jnp.clip(rby - lty, 0.0, None)
    inter = w * h
    union = (r_area + c_area) - inter
    iou = inter / jnp.maximum(union, 1e-9)

    ri = i * _BI + jax.lax.broadcasted_iota(jnp.int32, (_BI, 1), 0)
    cj = jax.lax.broadcasted_iota(jnp.int32, (1, _NPAD), 1)
    beats = (cs > rs) | ((cs == rs) & (cj < ri))

    suppressed = jnp.any((iou > _IOU_THRESH) & beats, axis=1, keepdims=True)
    out_ref[:, :] = jnp.where(suppressed, 0.0, rows)


def kernel(boxes, scores):
    data = jnp.zeros((_NPAD, 8), dtype=jnp.float32)
    data = data.at[:_N, 0:4].set(boxes)
    data = data.at[:_N, 4].set(scores)

    out = pl.pallas_call(
        _nms_block_kernel,
        grid=(_NPAD // _BI,),
        in_specs=[
            pl.BlockSpec((_BI, 8), lambda i: (i, 0)),
            pl.BlockSpec((8, _NPAD), lambda i: (0, 0)),
        ],
        out_specs=pl.BlockSpec((_BI, 8), lambda i: (i, 0)),
        out_shape=jax.ShapeDtypeStruct((_NPAD, 8), jnp.float32),
        compiler_params=pltpu.CompilerParams(
            dimension_semantics=("parallel",),
        ),
    )(data, data.T)

    return out[:_N, :5]
